# Initial kernel scaffold; baseline (speedup 1.0000x reference)
import jax
import jax.numpy as jnp
from jax.experimental import pallas as pl


def kernel(batch_confidence_maps, B, gauss_weight):
    raise NotImplementedError("write your pallas kernel here")


# speedup vs baseline: 27.5760x; 27.5760x over previous
import jax
import jax.numpy as jnp
from jax.experimental import pallas as pl
from jax.experimental.pallas import tpu as pltpu

_THRESHOLD = 0.01


def _map_kernel(gv_ref, gh_ref, x_ref, mask_ref, cnt_ref, pad_ref):
    i = pl.program_id(0)

    @pl.when(i == 0)
    def _init():
        pad_ref[...] = jnp.zeros_like(pad_ref)
        cnt_ref[0, 0] = 0.0

    m = jnp.maximum(x_ref[0, 0], x_ref[0, 1])
    s = 1.0 / (1.0 + jnp.exp(-m))
    pad_ref[pl.ds(2, 512), pl.ds(2, 512)] = s

    tmp = gv_ref[0] * pad_ref[pl.ds(0, 512), :]
    for r in range(1, 5):
        tmp = tmp + gv_ref[r] * pad_ref[pl.ds(r, 512), :]
    out = gh_ref[0] * tmp[:, 0:512]
    for c in range(1, 5):
        out = out + gh_ref[c] * tmp[:, c:c + 512]

    mask = jnp.where(out > _THRESHOLD, 1.0, 0.0)
    cnt_ref[0, 0] += jnp.sum(mask)
    is_first = (i % 5) == 0
    mask_ref[0, 0] = jnp.where(is_first, jnp.ones_like(mask), mask)


def kernel(batch_confidence_maps, B, gauss_weight):
    Bdim, L, A, H, W = batch_confidence_maps.shape
    N = Bdim * L
    x = batch_confidence_maps.reshape(N, A, H, W)
    g = gauss_weight.reshape(5, 5)
    gv = g[:, 2]
    gh = g[2, :] / g[2, 2]

    masks, cnt = pl.pallas_call(
        _map_kernel,
        grid=(N,),
        in_specs=[
            pl.BlockSpec(memory_space=pltpu.SMEM),
            pl.BlockSpec(memory_space=pltpu.SMEM),
            pl.BlockSpec((1, A, H, W), lambda i: (i, 0, 0, 0)),
        ],
        out_specs=[
            pl.BlockSpec((1, 1, H, W), lambda i: (i, 0, 0, 0)),
            pl.BlockSpec(memory_space=pltpu.SMEM),
        ],
        out_shape=[
            jax.ShapeDtypeStruct((N, 1, H, W), jnp.float32),
            jax.ShapeDtypeStruct((1, 1), jnp.float32),
        ],
        scratch_shapes=[pltpu.VMEM((H + 4, W + 4), jnp.float32)],
    )(gv, gh, x)

    rate = cnt[0, 0] / (N * H * W)
    return masks, rate
